# baseline (speedup 1.0000x reference)
import functools

import jax
import jax.numpy as jnp
from jax import lax
from jax.experimental import pallas as pl
from jax.experimental.pallas import tpu as pltpu
from jax.experimental.pallas import tpu_sc as plsc

_L = 16


@functools.lru_cache(maxsize=None)
def _make_kernel(T, D, V, S0, S1, S2):
    NW = 32
    CHUNK = T // NW
    NJ = CHUNK // 128
    GPJ = 128 // _L
    NQ = D // _L
    assert T % (NW * 128) == 0 and D % _L == 0
    HX, KX, LX = (S0 - 1) // 2, (S1 - 1) // 2, (S2 - 1) // 2

    mesh = plsc.VectorSubcoreMesh(core_axis_name="c", subcore_axis_name="s")

    @functools.partial(
        pl.kernel,
        mesh=mesh,
        out_type=jax.ShapeDtypeStruct((T, D), jnp.float32),
        scratch_types=[
            pltpu.VMEM((CHUNK,), jnp.int32),
            pltpu.VMEM((CHUNK,), jnp.int32),
            pltpu.VMEM((CHUNK,), jnp.int32),
            pltpu.VMEM((NJ, 128), jnp.int32),
            pltpu.VMEM((NJ, 128), jnp.int32),
            pltpu.VMEM((NJ, 128), jnp.int32),
            pltpu.VMEM((CHUNK,), jnp.float32),
            pltpu.VMEM((CHUNK, D), jnp.float32),
            pltpu.SemaphoreType.DMA,
            pltpu.SemaphoreType.DMA,
            pltpu.SemaphoreType.DMA,
            pltpu.SemaphoreType.DMA,
        ],
    )
    def body(src_hbm, hp_hbm, tab_hbm, out_hbm,
             a_v, b_v, c_v, lin_v, rid_v, rs_v, mf_v, rows_v,
             sem_in, sem_t, sem_r, sem_o):
        wid = lax.axis_index("s") * 2 + lax.axis_index("c")
        base = wid * CHUNK
        cps = [pltpu.async_copy(hp_hbm.at[pl.ds(base, CHUNK)], a_v, sem_in),
               pltpu.async_copy(hp_hbm.at[pl.ds(T + base, CHUNK)], b_v,
                                sem_in),
               pltpu.async_copy(hp_hbm.at[pl.ds(2 * T + base, CHUNK)], c_v,
                                sem_in)]
        for cp in cps:
            cp.wait()

        def stage1(g, j):
            og = g * _L
            a = a_v[pl.ds(og, _L)]
            b = b_v[pl.ds(og, _L)]
            c = c_v[pl.ds(og, _L)]
            inb = ((jnp.abs(a) <= HX) & (jnp.abs(b) <= KX)
                   & (jnp.abs(c) <= LX))
            ma = lax.rem(a, jnp.int32(S0))
            ma = jnp.where(ma < 0, ma + S0, ma)
            mb = lax.rem(b, jnp.int32(S1))
            mb = jnp.where(mb < 0, mb + S1, mb)
            mc = lax.rem(c, jnp.int32(S2))
            mc = jnp.where(mc < 0, mc + S2, mc)
            lin = ma * (S1 * S2) + mb * S2 + mc
            lin_v[j, pl.ds(og - j * 128, _L)] = lin
            a_v[pl.ds(og, _L)] = jnp.where(inb, 0, -1)
            return j

        tab_cps = []
        for j in range(NJ):
            lax.fori_loop(j * GPJ, (j + 1) * GPJ, stage1, j)
            tab_cps.append(
                pltpu.async_copy(tab_hbm.at[lin_v.at[j]], rid_v.at[j], sem_t))

        def sanitize(g, j):
            og = g * _L
            rid = rid_v[j, pl.ds(og - j * 128, _L)]
            val = (rid | a_v[pl.ds(og, _L)]) >= 0
            rs_v[j, pl.ds(og - j * 128, _L)] = jnp.where(val, rid, 0)
            mf_v[pl.ds(og, _L)] = jnp.where(val, jnp.float32(1.0),
                                            jnp.float32(0.0))
            return j

        row_cps = []
        for j in range(NJ):
            tab_cps[j].wait()
            lax.fori_loop(j * GPJ, (j + 1) * GPJ, sanitize, j)
            row_cps.append(
                pltpu.async_copy(src_hbm.at[rs_v.at[j]],
                                 rows_v.at[pl.ds(j * 128, 128)], sem_r))

        def mask_group(g, j):
            og = g * _L
            mf16 = mf_v[pl.ds(og, _L)]
            for r in range(_L):
                mf = jnp.broadcast_to(mf16[r], (_L,))
                for q in range(NQ):
                    rows_v[og + r, pl.ds(q * _L, _L)] = (
                        rows_v[og + r, pl.ds(q * _L, _L)] * mf)
            return j

        for j in range(NJ):
            row_cps[j].wait()
            lax.fori_loop(j * GPJ, (j + 1) * GPJ, mask_group, j)
        pltpu.async_copy(rows_v, out_hbm.at[pl.ds(base, CHUNK)],
                         sem_o).wait()

    return body


def kernel(source, H, cu_seqlens, reflection_id, Hmax):
    del cu_seqlens, Hmax
    T = H.shape[0]
    V, D = source.shape
    S0, S1, S2 = reflection_id.shape
    tab_flat = reflection_id.reshape(-1)
    Hi = H.astype(jnp.int32)
    hpack = jnp.concatenate([Hi[:, 0], Hi[:, 1], Hi[:, 2]])
    k = _make_kernel(T, D, V, S0, S1, S2)
    return k(source, hpack, tab_flat)


# speedup vs baseline: 1.0019x; 1.0019x over previous
import functools

import jax
import jax.numpy as jnp
from jax import lax
from jax.experimental import pallas as pl
from jax.experimental.pallas import tpu as pltpu
from jax.experimental.pallas import tpu_sc as plsc

_L = 16


@functools.lru_cache(maxsize=None)
def _make_kernel(T, D, V, S0, S1, S2):
    NW = 32
    CHUNK = T // NW
    CW = 64
    NJ = CHUNK // CW
    GPJ = CW // _L
    NQ = D // _L
    assert T % (NW * CW) == 0 and CW <= 128 and D % _L == 0
    HX, KX, LX = (S0 - 1) // 2, (S1 - 1) // 2, (S2 - 1) // 2

    mesh = plsc.VectorSubcoreMesh(core_axis_name="c", subcore_axis_name="s")

    @functools.partial(
        pl.kernel,
        mesh=mesh,
        out_type=jax.ShapeDtypeStruct((T, D), jnp.float32),
        scratch_types=[
            pltpu.VMEM((CHUNK,), jnp.int32),
            pltpu.VMEM((CHUNK,), jnp.int32),
            pltpu.VMEM((CHUNK,), jnp.int32),
            pltpu.VMEM((NJ, CW), jnp.int32),
            pltpu.VMEM((NJ, CW), jnp.int32),
            pltpu.VMEM((NJ, CW), jnp.int32),
            pltpu.VMEM((CHUNK,), jnp.float32),
            pltpu.VMEM((CHUNK, D), jnp.float32),
            pltpu.SemaphoreType.DMA,
            pltpu.SemaphoreType.DMA,
            pltpu.SemaphoreType.DMA,
            pltpu.SemaphoreType.DMA,
        ],
    )
    def body(src_hbm, hp_hbm, tab_hbm, out_hbm,
             a_v, b_v, c_v, lin_v, rid_v, rs_v, mf_v, rows_v,
             sem_in, sem_t, sem_r, sem_o):
        wid = lax.axis_index("s") * 2 + lax.axis_index("c")
        base = wid * CHUNK
        cps = [pltpu.async_copy(hp_hbm.at[pl.ds(base, CHUNK)], a_v, sem_in),
               pltpu.async_copy(hp_hbm.at[pl.ds(T + base, CHUNK)], b_v,
                                sem_in),
               pltpu.async_copy(hp_hbm.at[pl.ds(2 * T + base, CHUNK)], c_v,
                                sem_in)]
        for cp in cps:
            cp.wait()

        def stage1(g, j):
            og = g * _L
            a = a_v[pl.ds(og, _L)]
            b = b_v[pl.ds(og, _L)]
            c = c_v[pl.ds(og, _L)]
            inb = ((jnp.abs(a) <= HX) & (jnp.abs(b) <= KX)
                   & (jnp.abs(c) <= LX))
            ma = lax.rem(a, jnp.int32(S0))
            ma = jnp.where(ma < 0, ma + S0, ma)
            mb = lax.rem(b, jnp.int32(S1))
            mb = jnp.where(mb < 0, mb + S1, mb)
            mc = lax.rem(c, jnp.int32(S2))
            mc = jnp.where(mc < 0, mc + S2, mc)
            lin = ma * (S1 * S2) + mb * S2 + mc
            lin_v[j, pl.ds(og - j * CW, _L)] = lin
            a_v[pl.ds(og, _L)] = jnp.where(inb, 0, -1)
            return j

        tab_cps = []
        for j in range(NJ):
            lax.fori_loop(j * GPJ, (j + 1) * GPJ, stage1, j)
            tab_cps.append(
                pltpu.async_copy(tab_hbm.at[lin_v.at[j]], rid_v.at[j], sem_t))

        def sanitize(g, j):
            og = g * _L
            rid = rid_v[j, pl.ds(og - j * CW, _L)]
            val = (rid | a_v[pl.ds(og, _L)]) >= 0
            rs_v[j, pl.ds(og - j * CW, _L)] = jnp.where(val, rid, 0)
            mf_v[pl.ds(og, _L)] = jnp.where(val, jnp.float32(1.0),
                                            jnp.float32(0.0))
            return j

        row_cps = []
        for j in range(NJ):
            tab_cps[j].wait()
            lax.fori_loop(j * GPJ, (j + 1) * GPJ, sanitize, j)
            row_cps.append(
                pltpu.async_copy(src_hbm.at[rs_v.at[j]],
                                 rows_v.at[pl.ds(j * CW, CW)], sem_r))

        def mask_group(g, j):
            og = g * _L
            mf16 = mf_v[pl.ds(og, _L)]
            for r in range(_L):
                mf = jnp.broadcast_to(mf16[r], (_L,))
                for q in range(NQ):
                    rows_v[og + r, pl.ds(q * _L, _L)] = (
                        rows_v[og + r, pl.ds(q * _L, _L)] * mf)
            return j

        for j in range(NJ):
            row_cps[j].wait()
            lax.fori_loop(j * GPJ, (j + 1) * GPJ, mask_group, j)
        pltpu.async_copy(rows_v, out_hbm.at[pl.ds(base, CHUNK)],
                         sem_o).wait()

    return body


def kernel(source, H, cu_seqlens, reflection_id, Hmax):
    del cu_seqlens, Hmax
    T = H.shape[0]
    V, D = source.shape
    S0, S1, S2 = reflection_id.shape
    tab_flat = reflection_id.reshape(-1)
    hpack = H.astype(jnp.int32).T.reshape(-1)
    k = _make_kernel(T, D, V, S0, S1, S2)
    return k(source, hpack, tab_flat)


# speedup vs baseline: 1.0096x; 1.0077x over previous
import functools

import jax
import jax.numpy as jnp
from jax import lax
from jax.experimental import pallas as pl
from jax.experimental.pallas import tpu as pltpu
from jax.experimental.pallas import tpu_sc as plsc

_L = 16


@functools.lru_cache(maxsize=None)
def _make_kernel(T, D, V, S0, S1, S2):
    NW = 32
    CHUNK = T // NW
    NJ = CHUNK // 128
    GPJ = 128 // _L
    NQ = D // _L
    assert T % (NW * 128) == 0 and D % _L == 0
    HX, KX, LX = (S0 - 1) // 2, (S1 - 1) // 2, (S2 - 1) // 2

    mesh = plsc.VectorSubcoreMesh(core_axis_name="c", subcore_axis_name="s")

    @functools.partial(
        pl.kernel,
        mesh=mesh,
        out_type=jax.ShapeDtypeStruct((T, D), jnp.float32),
        scratch_types=[
            pltpu.VMEM((CHUNK,), jnp.int32),
            pltpu.VMEM((CHUNK,), jnp.int32),
            pltpu.VMEM((CHUNK,), jnp.int32),
            pltpu.VMEM((NJ, 128), jnp.int32),
            pltpu.VMEM((NJ, 128), jnp.int32),
            pltpu.VMEM((CHUNK,), jnp.float32),
            pltpu.VMEM((CHUNK, D), jnp.float32),
            pltpu.SemaphoreType.DMA,
            pltpu.SemaphoreType.DMA,
        ],
    )
    def body(src_hbm, hp_hbm, tab_hbm, out_hbm,
             a_v, b_v, c_v, lin_v, rid_v, mf_v, rows_v,
             sem_a, sem_b):
        wid = lax.axis_index("s") * 2 + lax.axis_index("c")
        base = wid * CHUNK
        cps = [pltpu.async_copy(hp_hbm.at[pl.ds(base, CHUNK)], a_v, sem_a),
               pltpu.async_copy(hp_hbm.at[pl.ds(T + base, CHUNK)], b_v,
                                sem_a),
               pltpu.async_copy(hp_hbm.at[pl.ds(2 * T + base, CHUNK)], c_v,
                                sem_a)]
        for cp in cps:
            cp.wait()

        def stage1(g, j):
            og = g * _L
            a = a_v[pl.ds(og, _L)]
            b = b_v[pl.ds(og, _L)]
            c = c_v[pl.ds(og, _L)]
            inb = ((jnp.abs(a) <= HX) & (jnp.abs(b) <= KX)
                   & (jnp.abs(c) <= LX))
            ma = lax.rem(a, jnp.int32(S0))
            ma = jnp.where(ma < 0, ma + S0, ma)
            mb = lax.rem(b, jnp.int32(S1))
            mb = jnp.where(mb < 0, mb + S1, mb)
            mc = lax.rem(c, jnp.int32(S2))
            mc = jnp.where(mc < 0, mc + S2, mc)
            lin = ma * (S1 * S2) + mb * S2 + mc
            lin_v[j, pl.ds(og - j * 128, _L)] = lin
            a_v[pl.ds(og, _L)] = jnp.where(inb, 0, -1)
            return j

        tab_cps = []
        for j in range(NJ):
            lax.fori_loop(j * GPJ, (j + 1) * GPJ, stage1, j)
            tab_cps.append(
                pltpu.async_copy(tab_hbm.at[lin_v.at[j]], rid_v.at[j], sem_a))

        def sanitize(g, j):
            og = g * _L
            rid = rid_v[j, pl.ds(og - j * 128, _L)]
            val = (rid | a_v[pl.ds(og, _L)]) >= 0
            lin_v[j, pl.ds(og - j * 128, _L)] = jnp.where(val, rid, 0)
            mf_v[pl.ds(og, _L)] = jnp.where(val, jnp.float32(1.0),
                                            jnp.float32(0.0))
            return j

        row_cps = []
        for j in range(NJ):
            tab_cps[j].wait()
            lax.fori_loop(j * GPJ, (j + 1) * GPJ, sanitize, j)
            row_cps.append(
                pltpu.async_copy(src_hbm.at[lin_v.at[j]],
                                 rows_v.at[pl.ds(j * 128, 128)], sem_b))

        def mask_group(g, j):
            og = g * _L
            mf16 = mf_v[pl.ds(og, _L)]
            for r in range(_L):
                mf = jnp.broadcast_to(mf16[r], (_L,))
                for q in range(NQ):
                    rows_v[og + r, pl.ds(q * _L, _L)] = (
                        rows_v[og + r, pl.ds(q * _L, _L)] * mf)
            return j

        for j in range(NJ):
            row_cps[j].wait()
            lax.fori_loop(j * GPJ, (j + 1) * GPJ, mask_group, j)
        pltpu.async_copy(rows_v, out_hbm.at[pl.ds(base, CHUNK)],
                         sem_b).wait()

    return body


def kernel(source, H, cu_seqlens, reflection_id, Hmax):
    del cu_seqlens, Hmax
    T = H.shape[0]
    V, D = source.shape
    S0, S1, S2 = reflection_id.shape
    tab_flat = reflection_id.reshape(-1)
    hpack = H.astype(jnp.int32).T.reshape(-1)
    k = _make_kernel(T, D, V, S0, S1, S2)
    return k(source, hpack, tab_flat)
